# baseline (speedup 1.0000x reference)
import functools

import jax
import jax.numpy as jnp
from jax import lax
from jax.experimental import pallas as pl
from jax.experimental.pallas import tpu as pltpu
from jax.experimental.pallas import tpu_sc as plsc

SY, SX = 7, 7
P_ROI = SY * SX
K = 4
ROW_PAD = 200
HALF = ROW_PAD // 2
NW = 32
LANES = 16
T_PER_W = 160


def _make_idx_w(boxes, assoc, H, W):
    n = boxes.shape[0]
    sy, sx = boxes[:, 0, 0], boxes[:, 0, 1]
    ey, ex = boxes[:, 1, 0], boxes[:, 1, 1]
    j7 = jnp.arange(SY, dtype=boxes.dtype)
    cc_y = jnp.minimum(j7 * ((ey - sy) / (SY - 1))[:, None] + sy[:, None], ey[:, None])
    cc_x = jnp.minimum(j7 * ((ex - sx) / (SX - 1))[:, None] + sx[:, None], ex[:, None])
    fy, fx = jnp.floor(cc_y), jnp.floor(cc_x)
    y_i = jnp.stack([fy, jnp.ceil(cc_y)], 1).astype(jnp.int32)
    x_i = jnp.stack([fx, jnp.ceil(cc_x)], 1).astype(jnp.int32)
    wy2, wx2 = cc_y - fy, cc_x - fx
    wy = jnp.stack([1.0 - wy2, wy2], 1)
    wx = jnp.stack([1.0 - wx2, wx2], 1)
    yterm = assoc[:, None, None] * (H * W) + y_i * W
    idx = (yterm.transpose(0, 2, 1)[:, :, None, :, None]
           + x_i.transpose(0, 2, 1)[:, None, :, None, :]).reshape(n, P_ROI * K)
    w = (wy.transpose(0, 2, 1)[:, :, None, :, None]
         * wx.transpose(0, 2, 1)[:, None, :, None, :]).reshape(n, P_ROI * K)
    idx_p = jnp.zeros((n, ROW_PAD), jnp.int32).at[:, : P_ROI * K].set(idx)
    w_p = jnp.zeros((n, ROW_PAD), jnp.float32).at[:, : P_ROI * K].set(w)
    return idx_p.reshape(n, 2, HALF), w_p


def _build_sc_call(n, c):
    mesh = plsc.VectorSubcoreMesh(core_axis_name="c", subcore_axis_name="s")
    n_last = T_PER_W // 4 - 1
    scratch = (
        [pltpu.VMEM((2, HALF), jnp.int32) for _ in range(4)]
        + [pltpu.VMEM((ROW_PAD + LANES,), jnp.float32) for _ in range(4)]
        + [pltpu.VMEM((2, HALF, c), jnp.float32) for _ in range(2)]
        + [pltpu.VMEM((c * P_ROI,), jnp.float32) for _ in range(2)]
        + [pltpu.SemaphoreType.DMA for _ in range(8)]
    )

    @functools.partial(
        pl.kernel,
        out_type=jax.ShapeDtypeStruct((n * c * P_ROI,), jnp.float32),
        mesh=mesh,
        scratch_types=scratch,
        compiler_params=pltpu.CompilerParams(
            needs_layout_passes=False, use_tc_tiling_on_sc=False
        ),
    )
    def sc_roi_align(table, idxs, ws, out, *scr):
        idx_b, w_b = scr[0:4], scr[4:8]
        rows_b, out_b = scr[8:10], scr[10:12]
        siw, sg, so = scr[12:16], scr[16:18], scr[18:20]
        nc = plsc.get_sparse_core_info().num_cores
        wid = lax.axis_index("s") * nc + lax.axis_index("c")
        base = jnp.minimum(wid * T_PER_W, n - T_PER_W)

        def start_iw(t, r):
            pltpu.async_copy(idxs.at[base + t], idx_b[r], siw[r])
            pltpu.async_copy(ws.at[base + t], w_b[r].at[pl.ds(0, ROW_PAD)], siw[r])

        def wait_iw(t, r):
            pltpu.make_async_copy(idxs.at[base + t], idx_b[r], siw[r]).wait()
            pltpu.make_async_copy(
                ws.at[base + t], w_b[r].at[pl.ds(0, ROW_PAD)], siw[r]
            ).wait()

        def start_gather(ri, rg):
            pltpu.async_copy(table.at[idx_b[ri].at[0]], rows_b[rg].at[0], sg[rg])
            pltpu.async_copy(table.at[idx_b[ri].at[1]], rows_b[rg].at[1], sg[rg])

        def wait_gather(ri, rg):
            pltpu.make_async_copy(table.at[idx_b[ri].at[0]], rows_b[rg].at[0], sg[rg]).wait()
            pltpu.make_async_copy(table.at[idx_b[ri].at[1]], rows_b[rg].at[1], sg[rg]).wait()

        def start_out(t, ro):
            pltpu.async_copy(
                out_b[ro], out.at[pl.ds((base + t) * c * P_ROI, c * P_ROI)], so[ro]
            )

        def wait_out(t, ro):
            pltpu.make_async_copy(
                out_b[ro], out.at[pl.ds((base + t) * c * P_ROI, c * P_ROI)], so[ro]
            ).wait()

        def compute(ri, rg, ro):
            w_v, rows_v, out_v = w_b[ri], rows_b[rg], out_b[ro]
            cidx49 = [
                (lax.iota(jnp.int32, LANES) + c0) * P_ROI for c0 in range(0, c, LANES)
            ]
            for half in range(2):
                pts = HALF // K if half == 0 else P_ROI - HALF // K

                def pt_body(p, carry2):
                    wvec = [
                        plsc.load_gather(
                            w_v,
                            [jnp.full((LANES,), half * HALF + p * K + kk, jnp.int32)],
                        )
                        for kk in range(K)
                    ]
                    pg16 = jnp.full((LANES,), half * (HALF // K) + p, jnp.int32)
                    for ci, c0 in enumerate(range(0, c, LANES)):
                        acc = rows_v[half, p * K, pl.ds(c0, LANES)] * wvec[0]
                        for kk in range(1, K):
                            acc = acc + rows_v[half, p * K + kk, pl.ds(c0, LANES)] * wvec[kk]
                        plsc.store_scatter(out_v, [cidx49[ci] + pg16], acc)
                    return carry2

                lax.fori_loop(0, pts, pt_body, 0)

        for r in range(4):
            start_iw(r, r)
        wait_iw(0, 0)
        start_gather(0, 0)

        def j_body(j, carry):
            for u in range(4):
                t = 4 * j + u
                g = u & 1
                wait_gather(u & 3, g)
                if u < 3:
                    wait_iw(t + 1, (u + 1) & 3)
                    start_gather((u + 1) & 3, g ^ 1)
                else:
                    @pl.when(j < n_last)
                    def _():
                        wait_iw(t + 1, (u + 1) & 3)
                        start_gather((u + 1) & 3, g ^ 1)
                if u >= 2:
                    wait_out(t - 2, g)
                else:
                    @pl.when(j > 0)
                    def _():
                        wait_out(t - 2, g)
                compute(u & 3, g, g)
                start_out(t, g)

                @pl.when(j < n_last)
                def _():
                    start_iw(t + 4, u)
            return carry

        lax.fori_loop(0, T_PER_W // 4, j_body, 0)
        wait_out(T_PER_W - 2, 0)
        wait_out(T_PER_W - 1, 1)

    return sc_roi_align


def kernel(featuremap, boxes, box_sample_association):
    b, c, h, w = featuremap.shape
    n = boxes.shape[0]
    table = jnp.transpose(featuremap, (0, 2, 3, 1)).reshape(b * h * w, c)
    idx, wts = _make_idx_w(boxes, box_sample_association, h, w)
    out = _build_sc_call(n, c)(table, idx, wts)
    return out.reshape(n, c, SY, SX)


# speedup vs baseline: 1.8127x; 1.8127x over previous
import functools

import jax
import jax.numpy as jnp
from jax import lax
from jax.experimental import pallas as pl
from jax.experimental.pallas import tpu as pltpu
from jax.experimental.pallas import tpu_sc as plsc

SY, SX = 7, 7
P_ROI = SY * SX
K = 4
ROW_PAD = 200
HALF = ROW_PAD // 2
NW = 32
LANES = 16
T_PER_W = 160


def _make_idx_w(boxes, assoc, H, W):
    n = boxes.shape[0]
    sy, sx = boxes[:, 0, 0], boxes[:, 0, 1]
    ey, ex = boxes[:, 1, 0], boxes[:, 1, 1]
    j7 = jnp.arange(SY, dtype=boxes.dtype)
    cc_y = jnp.minimum(j7 * ((ey - sy) / (SY - 1))[:, None] + sy[:, None], ey[:, None])
    cc_x = jnp.minimum(j7 * ((ex - sx) / (SX - 1))[:, None] + sx[:, None], ex[:, None])
    fy, fx = jnp.floor(cc_y), jnp.floor(cc_x)
    y_i = jnp.stack([fy, jnp.ceil(cc_y)], 1).astype(jnp.int32)
    x_i = jnp.stack([fx, jnp.ceil(cc_x)], 1).astype(jnp.int32)
    wy2, wx2 = cc_y - fy, cc_x - fx
    wy = jnp.stack([1.0 - wy2, wy2], 1)
    wx = jnp.stack([1.0 - wx2, wx2], 1)
    yterm = assoc[:, None, None] * (H * W) + y_i * W
    idx = (yterm.transpose(0, 2, 1)[:, :, None, :, None]
           + x_i.transpose(0, 2, 1)[:, None, :, None, :]).reshape(n, P_ROI * K)
    w = (wy.transpose(0, 2, 1)[:, :, None, :, None]
         * wx.transpose(0, 2, 1)[:, None, :, None, :]).reshape(n, P_ROI * K)
    idx_p = jnp.zeros((n, ROW_PAD), jnp.int32).at[:, : P_ROI * K].set(idx)
    w_p = jnp.zeros((n, ROW_PAD), jnp.float32).at[:, : P_ROI * K].set(w)
    return idx_p.reshape(n, 2, HALF), w_p


def _build_sc_call(n, c):
    mesh = plsc.VectorSubcoreMesh(core_axis_name="c", subcore_axis_name="s")
    n_last = T_PER_W // 4 - 1
    scratch = (
        [pltpu.VMEM((2, HALF), jnp.int32) for _ in range(4)]
        + [pltpu.VMEM((ROW_PAD + LANES,), jnp.float32) for _ in range(4)]
        + [pltpu.VMEM((2, HALF, c), jnp.float32) for _ in range(2)]
        + [pltpu.VMEM((c, P_ROI), jnp.float32) for _ in range(2)]
        + [pltpu.SemaphoreType.DMA for _ in range(8)]
    )

    @functools.partial(
        pl.kernel,
        out_type=jax.ShapeDtypeStruct((n, c, P_ROI), jnp.float32),
        mesh=mesh,
        scratch_types=scratch,
        compiler_params=pltpu.CompilerParams(
            needs_layout_passes=False, use_tc_tiling_on_sc=False
        ),
    )
    def sc_roi_align(table, idxs, ws, out, *scr):
        idx_b, w_b = scr[0:4], scr[4:8]
        rows_b, out_b = scr[8:10], scr[10:12]
        siw, sg, so = scr[12:16], scr[16:18], scr[18:20]
        nc = plsc.get_sparse_core_info().num_cores
        wid = lax.axis_index("s") * nc + lax.axis_index("c")
        base = jnp.minimum(wid * T_PER_W, n - T_PER_W)

        def start_iw(t, r):
            pltpu.async_copy(idxs.at[base + t], idx_b[r], siw[r])
            pltpu.async_copy(ws.at[base + t], w_b[r].at[pl.ds(0, ROW_PAD)], siw[r])

        def wait_iw(t, r):
            pltpu.make_async_copy(idxs.at[base + t], idx_b[r], siw[r]).wait()
            pltpu.make_async_copy(
                ws.at[base + t], w_b[r].at[pl.ds(0, ROW_PAD)], siw[r]
            ).wait()

        def start_gather(ri, rg):
            pltpu.async_copy(table.at[idx_b[ri].at[0]], rows_b[rg].at[0], sg[rg])
            pltpu.async_copy(table.at[idx_b[ri].at[1]], rows_b[rg].at[1], sg[rg])

        def wait_gather(ri, rg):
            pltpu.make_async_copy(table.at[idx_b[ri].at[0]], rows_b[rg].at[0], sg[rg]).wait()
            pltpu.make_async_copy(table.at[idx_b[ri].at[1]], rows_b[rg].at[1], sg[rg]).wait()

        def start_out(t, ro):
            pltpu.async_copy(out_b[ro], out.at[base + t], so[ro])

        def wait_out(t, ro):
            pltpu.make_async_copy(out_b[ro], out.at[base + t], so[ro]).wait()

        def compute(ri, rg, ro):
            w_v, rows_v, out_v = w_b[ri], rows_b[rg], out_b[ro]
            cidx = [lax.iota(jnp.int32, LANES) + c0 for c0 in range(0, c, LANES)]
            for half in range(2):
                pts = HALF // K if half == 0 else P_ROI - HALF // K

                def pt_body(p, carry2):
                    wvec = [
                        plsc.load_gather(
                            w_v,
                            [jnp.full((LANES,), half * HALF + p * K + kk, jnp.int32)],
                        )
                        for kk in range(K)
                    ]
                    pg16 = jnp.full((LANES,), half * (HALF // K) + p, jnp.int32)
                    for ci, c0 in enumerate(range(0, c, LANES)):
                        acc = rows_v[half, p * K, pl.ds(c0, LANES)] * wvec[0]
                        for kk in range(1, K):
                            acc = acc + rows_v[half, p * K + kk, pl.ds(c0, LANES)] * wvec[kk]
                        plsc.store_scatter(out_v, [cidx[ci], pg16], acc)
                    return carry2

                lax.fori_loop(0, pts, pt_body, 0)

        for r in range(4):
            start_iw(r, r)
        wait_iw(0, 0)
        start_gather(0, 0)

        def j_body(j, carry):
            for u in range(4):
                t = 4 * j + u
                g = u & 1
                wait_gather(u & 3, g)
                if u < 3:
                    wait_iw(t + 1, (u + 1) & 3)
                    start_gather((u + 1) & 3, g ^ 1)
                else:
                    @pl.when(j < n_last)
                    def _():
                        wait_iw(t + 1, (u + 1) & 3)
                        start_gather((u + 1) & 3, g ^ 1)
                if u >= 2:
                    wait_out(t - 2, g)
                else:
                    @pl.when(j > 0)
                    def _():
                        wait_out(t - 2, g)
                compute(u & 3, g, g)
                start_out(t, g)

                @pl.when(j < n_last)
                def _():
                    start_iw(t + 4, u)
            return carry

        lax.fori_loop(0, T_PER_W // 4, j_body, 0)
        wait_out(T_PER_W - 2, 0)
        wait_out(T_PER_W - 1, 1)

    return sc_roi_align


def kernel(featuremap, boxes, box_sample_association):
    b, c, h, w = featuremap.shape
    n = boxes.shape[0]
    table = jnp.transpose(featuremap, (0, 2, 3, 1)).reshape(b * h * w, c)
    idx, wts = _make_idx_w(boxes, box_sample_association, h, w)
    out = _build_sc_call(n, c)(table, idx, wts)
    return out.reshape(n, c, SY, SX)


# speedup vs baseline: 1.8245x; 1.0065x over previous
import functools

import jax
import jax.numpy as jnp
from jax import lax
from jax.experimental import pallas as pl
from jax.experimental.pallas import tpu as pltpu
from jax.experimental.pallas import tpu_sc as plsc

SY, SX = 7, 7
P_ROI = SY * SX
K = 4
ROW_PAD = 200
HALF = ROW_PAD // 2
NW = 32
LANES = 16
T_PER_W = 160


def _make_idx_w(boxes, assoc, H, W):
    n = boxes.shape[0]
    sy, sx = boxes[:, 0, 0], boxes[:, 0, 1]
    ey, ex = boxes[:, 1, 0], boxes[:, 1, 1]
    j7 = jnp.arange(SY, dtype=boxes.dtype)
    cc_y = jnp.minimum(j7 * ((ey - sy) / (SY - 1))[:, None] + sy[:, None], ey[:, None])
    cc_x = jnp.minimum(j7 * ((ex - sx) / (SX - 1))[:, None] + sx[:, None], ex[:, None])
    fy, fx = jnp.floor(cc_y), jnp.floor(cc_x)
    y_i = jnp.stack([fy, jnp.ceil(cc_y)], 1).astype(jnp.int32)
    x_i = jnp.stack([fx, jnp.ceil(cc_x)], 1).astype(jnp.int32)
    wy2, wx2 = cc_y - fy, cc_x - fx
    wy = jnp.stack([1.0 - wy2, wy2], 1)
    wx = jnp.stack([1.0 - wx2, wx2], 1)
    yterm = assoc[:, None, None] * (H * W) + y_i * W
    idx = (yterm.transpose(0, 2, 1)[:, :, None, :, None]
           + x_i.transpose(0, 2, 1)[:, None, :, None, :]).reshape(n, P_ROI * K)
    w = (wy.transpose(0, 2, 1)[:, :, None, :, None]
         * wx.transpose(0, 2, 1)[:, None, :, None, :]).reshape(n, P_ROI * K)
    idx_p = jnp.zeros((n, ROW_PAD), jnp.int32).at[:, : P_ROI * K].set(idx)
    w_p = jnp.zeros((n, ROW_PAD), jnp.float32).at[:, : P_ROI * K].set(w)
    return idx_p.reshape(n, 2, HALF), w_p


def _build_sc_call(n, c):
    mesh = plsc.VectorSubcoreMesh(core_axis_name="c", subcore_axis_name="s")
    n_last = T_PER_W // 4 - 1
    scratch = (
        [pltpu.VMEM((2, HALF), jnp.int32) for _ in range(4)]
        + [pltpu.VMEM((ROW_PAD + LANES,), jnp.float32) for _ in range(4)]
        + [pltpu.VMEM((2, HALF, c), jnp.float32) for _ in range(2)]
        + [pltpu.VMEM((c, P_ROI), jnp.float32) for _ in range(2)]
        + [pltpu.SemaphoreType.DMA for _ in range(8)]
    )

    @functools.partial(
        pl.kernel,
        out_type=jax.ShapeDtypeStruct((n, c, P_ROI), jnp.float32),
        mesh=mesh,
        scratch_types=scratch,
        compiler_params=pltpu.CompilerParams(
            needs_layout_passes=False, use_tc_tiling_on_sc=False
        ),
    )
    def sc_roi_align(table, idxs, ws, out, *scr):
        idx_b, w_b = scr[0:4], scr[4:8]
        rows_b, out_b = scr[8:10], scr[10:12]
        siw, sg, so = scr[12:16], scr[16:18], scr[18:20]
        nc = plsc.get_sparse_core_info().num_cores
        wid = lax.axis_index("s") * nc + lax.axis_index("c")
        base = jnp.minimum(wid * T_PER_W, n - T_PER_W)

        def start_iw(t, r):
            pltpu.async_copy(idxs.at[base + t], idx_b[r], siw[r])
            pltpu.async_copy(ws.at[base + t], w_b[r].at[pl.ds(0, ROW_PAD)], siw[r])

        def wait_iw(t, r):
            pltpu.make_async_copy(idxs.at[base + t], idx_b[r], siw[r]).wait()
            pltpu.make_async_copy(
                ws.at[base + t], w_b[r].at[pl.ds(0, ROW_PAD)], siw[r]
            ).wait()

        def start_gather(ri, rg):
            pltpu.async_copy(table.at[idx_b[ri].at[0]], rows_b[rg].at[0], sg[rg])
            pltpu.async_copy(table.at[idx_b[ri].at[1]], rows_b[rg].at[1], sg[rg])

        def wait_gather(ri, rg):
            pltpu.make_async_copy(table.at[idx_b[ri].at[0]], rows_b[rg].at[0], sg[rg]).wait()
            pltpu.make_async_copy(table.at[idx_b[ri].at[1]], rows_b[rg].at[1], sg[rg]).wait()

        def start_out(t, ro):
            pltpu.async_copy(out_b[ro], out.at[base + t], so[ro])

        def wait_out(t, ro):
            pltpu.make_async_copy(out_b[ro], out.at[base + t], so[ro]).wait()

        def compute(ri, rg, ro):
            w_v, rows_v, out_v = w_b[ri], rows_b[rg], out_b[ro]
            cidx = [lax.iota(jnp.int32, LANES) + c0 for c0 in range(0, c, LANES)]
            for half in range(2):
                pts = HALF // K if half == 0 else P_ROI - HALF // K

                def pt_body(p, carry2):
                    wvec = [
                        plsc.load_gather(
                            w_v,
                            [jnp.full((LANES,), half * HALF + p * K + kk, jnp.int32)],
                        )
                        for kk in range(K)
                    ]
                    pg16 = jnp.full((LANES,), half * (HALF // K) + p, jnp.int32)
                    for ci in range(0, c // LANES, 2):
                        r = [
                            rows_v[half, p * K + kk, pl.ds(ci * LANES, LANES)]
                            for kk in range(K)
                        ]
                        s = [
                            rows_v[half, p * K + kk, pl.ds((ci + 1) * LANES, LANES)]
                            for kk in range(K)
                        ]
                        acc0 = (r[0] * wvec[0] + r[1] * wvec[1]) + (
                            r[2] * wvec[2] + r[3] * wvec[3]
                        )
                        acc1 = (s[0] * wvec[0] + s[1] * wvec[1]) + (
                            s[2] * wvec[2] + s[3] * wvec[3]
                        )
                        plsc.store_scatter(out_v, [cidx[ci], pg16], acc0)
                        plsc.store_scatter(out_v, [cidx[ci + 1], pg16], acc1)
                    return carry2

                lax.fori_loop(0, pts, pt_body, 0)

        for r in range(4):
            start_iw(r, r)
        wait_iw(0, 0)
        start_gather(0, 0)

        def j_body(j, carry):
            for u in range(4):
                t = 4 * j + u
                g = u & 1
                wait_gather(u & 3, g)
                if u < 3:
                    wait_iw(t + 1, (u + 1) & 3)
                    start_gather((u + 1) & 3, g ^ 1)
                else:
                    @pl.when(j < n_last)
                    def _():
                        wait_iw(t + 1, (u + 1) & 3)
                        start_gather((u + 1) & 3, g ^ 1)
                if u >= 2:
                    wait_out(t - 2, g)
                else:
                    @pl.when(j > 0)
                    def _():
                        wait_out(t - 2, g)
                compute(u & 3, g, g)
                start_out(t, g)

                @pl.when(j < n_last)
                def _():
                    start_iw(t + 4, u)
            return carry

        lax.fori_loop(0, T_PER_W // 4, j_body, 0)
        wait_out(T_PER_W - 2, 0)
        wait_out(T_PER_W - 1, 1)

    return sc_roi_align


def kernel(featuremap, boxes, box_sample_association):
    b, c, h, w = featuremap.shape
    n = boxes.shape[0]
    table = jnp.transpose(featuremap, (0, 2, 3, 1)).reshape(b * h * w, c)
    idx, wts = _make_idx_w(boxes, box_sample_association, h, w)
    out = _build_sc_call(n, c)(table, idx, wts)
    return out.reshape(n, c, SY, SX)
